# Initial kernel scaffold; baseline (speedup 1.0000x reference)
import jax
import jax.numpy as jnp
from jax.experimental import pallas as pl


def kernel(X, theta_v, reference_pts, w):
    raise NotImplementedError("write your pallas kernel here")


# speedup vs baseline: 2.9528x; 2.9528x over previous
import functools

import numpy as np
import jax
import jax.numpy as jnp
from jax.experimental import pallas as pl
from jax.experimental.pallas import tpu as pltpu


def _interp_matrix(n: int, m: int) -> np.ndarray:
    j = np.arange(m, dtype=np.int64)
    num = (j + 1) * (n + 1)
    count = (num - 1) // (m + 1)
    ind = np.clip(count - 1, 0, n - 2)
    x_ind = (ind + 1) / (n + 1)
    xnew = (j + 1) / (m + 1)
    dx = 1.0 / (n + 1)
    eps = float(np.finfo(np.float32).eps)
    t = (xnew - x_ind) / (eps + dx)
    a = np.zeros((m, n), dtype=np.float64)
    np.add.at(a, (j, ind), 1.0 - t)
    np.add.at(a, (j, ind + 1), t)
    return a.astype(np.float32)


def _bitonic_sort_axis0(x):
    n = x.shape[0]
    row = jax.lax.broadcasted_iota(jnp.int32, x.shape, 0)
    k = 2
    while k <= n:
        asc = (row & k) == 0
        j = k // 2
        while j >= 1:
            low = (row & j) == 0
            xm = jnp.roll(x, -j, axis=0)
            xp = jnp.roll(x, j, axis=0)
            partner = jnp.where(low, xm, xp)
            want_min = low == asc
            x = jnp.where(want_min, jnp.minimum(x, partner),
                          jnp.maximum(x, partner))
            j //= 2
        k *= 2
    return x


def _body(x_ref, th_ref, ref_ref, w_ref, a_ref, out_ref):
    th = th_ref[...]
    inv_norm = jax.lax.rsqrt(jnp.sum(th * th, axis=1, keepdims=True))
    wn = th * inv_norm
    proj = jax.lax.dot_general(
        x_ref[0], wn, (((1,), (1,)), ((), ())),
        preferred_element_type=jnp.float32)
    srt = _bitonic_sort_axis0(proj)
    w = w_ref[...]
    wa = jax.lax.dot_general(
        w, a_ref[...], (((1,), (0,)), ((), ())),
        preferred_element_type=jnp.float32)
    red = jax.lax.dot_general(
        wa, srt, (((1,), (0,)), ((), ())),
        preferred_element_type=jnp.float32)
    cst = jax.lax.dot_general(
        w, ref_ref[...], (((1,), (0,)), ((), ())),
        preferred_element_type=jnp.float32)
    out_ref[...] = (cst - red)[None]


def kernel(X, theta_v, reference_pts, w):
    b, n, d = X.shape
    s = theta_v.shape[0]
    m = reference_pts.shape[0]
    a_mat = jnp.asarray(_interp_matrix(n, m))

    s_blk = 256
    grid = (b, s // s_blk)

    out3 = pl.pallas_call(
        _body,
        grid=grid,
        in_specs=[
            pl.BlockSpec((1, n, d), lambda i, j: (i, 0, 0)),
            pl.BlockSpec((s_blk, d), lambda i, j: (j, 0)),
            pl.BlockSpec((m, s_blk), lambda i, j: (0, j)),
            pl.BlockSpec((1, m), lambda i, j: (0, 0)),
            pl.BlockSpec((m, n), lambda i, j: (0, 0)),
        ],
        out_specs=pl.BlockSpec((1, 1, s_blk), lambda i, j: (i, 0, j)),
        out_shape=jax.ShapeDtypeStruct((b, 1, s), jnp.float32),
        compiler_params=pltpu.CompilerParams(
            dimension_semantics=("parallel", "parallel"),
        ),
    )(X, theta_v, reference_pts, w, a_mat)
    return out3.reshape(b, s)


# speedup vs baseline: 6.0824x; 2.0599x over previous
import functools

import numpy as np
import jax
import jax.numpy as jnp
from jax.experimental import pallas as pl
from jax.experimental.pallas import tpu as pltpu


def _interp_matrix(n: int, m: int) -> np.ndarray:
    j = np.arange(m, dtype=np.int64)
    num = (j + 1) * (n + 1)
    count = (num - 1) // (m + 1)
    ind = np.clip(count - 1, 0, n - 2)
    x_ind = (ind + 1) / (n + 1)
    xnew = (j + 1) / (m + 1)
    dx = 1.0 / (n + 1)
    eps = float(np.finfo(np.float32).eps)
    t = (xnew - x_ind) / (eps + dx)
    a = np.zeros((m, n), dtype=np.float64)
    np.add.at(a, (j, ind), 1.0 - t)
    np.add.at(a, (j, ind + 1), t)
    return a.astype(np.float32)


def _bit_reverse_perm(n: int) -> np.ndarray:
    bits = n.bit_length() - 1
    p = np.arange(n)
    r = np.zeros(n, dtype=np.int64)
    for b in range(bits):
        r |= ((p >> b) & 1) << (bits - 1 - b)
    return r


def _cmpex_roll(x, row, jp, kp):
    low = (row & jp) == 0
    xm = jnp.roll(x, -jp, axis=0)
    xp = jnp.roll(x, jp, axis=0)
    partner = jnp.where(low, xm, xp)
    if kp is None:
        want_min = low
    else:
        asc = (row & kp) == 0
        want_min = low == asc
    return jnp.where(want_min, jnp.minimum(x, partner),
                     jnp.maximum(x, partner))


def _cmpex_reshape(x, row, jp, kp):
    n, s = x.shape
    g = n // (2 * jp)
    xr = x.reshape(g, 2, jp, s)
    a = xr[:, 0]
    b = xr[:, 1]
    mn = jnp.minimum(a, b)
    mx = jnp.maximum(a, b)
    if kp is None:
        na, nb = mn, mx
    else:
        asc = ((row & kp) == 0).reshape(g, 2, jp, 1)[:, 0]
        na = jnp.where(asc, mn, mx)
        nb = jnp.where(asc, mx, mn)
    return jnp.concatenate([na[:, None], nb[:, None]], axis=1).reshape(n, s)


def _bitonic_sort_bitrev(x):
    n = x.shape[0]
    bits = n.bit_length() - 1
    row = jax.lax.broadcasted_iota(jnp.int32, (n, 1), 0)
    for a in range(1, bits + 1):
        kp = None if a == bits else (1 << (bits - 1 - a))
        for b in range(a - 1, -1, -1):
            jp = 1 << (bits - 1 - b)
            if jp >= 8:
                x = _cmpex_reshape(x, row, jp, kp)
            else:
                x = _cmpex_roll(x, row, jp, kp)
    return x


def _body(x_ref, th_ref, ref_ref, w_ref, a_ref, out_ref):
    th = th_ref[...]
    inv_norm = jax.lax.rsqrt(jnp.sum(th * th, axis=1, keepdims=True))
    wn = th * inv_norm
    proj = jax.lax.dot_general(
        x_ref[0], wn, (((1,), (1,)), ((), ())),
        preferred_element_type=jnp.float32)
    srt = _bitonic_sort_bitrev(proj)
    w = w_ref[...]
    wa = jax.lax.dot_general(
        w, a_ref[...], (((1,), (0,)), ((), ())),
        preferred_element_type=jnp.float32)
    red = jax.lax.dot_general(
        wa, srt, (((1,), (0,)), ((), ())),
        preferred_element_type=jnp.float32)
    cst = jax.lax.dot_general(
        w, ref_ref[...], (((1,), (0,)), ((), ())),
        preferred_element_type=jnp.float32)
    out_ref[...] = (cst - red)[None]


def kernel(X, theta_v, reference_pts, w):
    b, n, d = X.shape
    s = theta_v.shape[0]
    m = reference_pts.shape[0]
    a_np = _interp_matrix(n, m)
    rev = _bit_reverse_perm(n)
    a_mat = jnp.asarray(a_np[:, rev])

    s_blk = 256
    grid = (b, s // s_blk)

    out3 = pl.pallas_call(
        _body,
        grid=grid,
        in_specs=[
            pl.BlockSpec((1, n, d), lambda i, j: (i, 0, 0)),
            pl.BlockSpec((s_blk, d), lambda i, j: (j, 0)),
            pl.BlockSpec((m, s_blk), lambda i, j: (0, j)),
            pl.BlockSpec((1, m), lambda i, j: (0, 0)),
            pl.BlockSpec((m, n), lambda i, j: (0, 0)),
        ],
        out_specs=pl.BlockSpec((1, 1, s_blk), lambda i, j: (i, 0, j)),
        out_shape=jax.ShapeDtypeStruct((b, 1, s), jnp.float32),
        compiler_params=pltpu.CompilerParams(
            dimension_semantics=("parallel", "parallel"),
        ),
    )(X, theta_v, reference_pts, w, a_mat)
    return out3.reshape(b, s)


# speedup vs baseline: 7.5559x; 1.2423x over previous
import functools

import numpy as np
import jax
import jax.numpy as jnp
from jax.experimental import pallas as pl
from jax.experimental.pallas import tpu as pltpu


def _interp_matrix(n: int, m: int) -> np.ndarray:
    j = np.arange(m, dtype=np.int64)
    num = (j + 1) * (n + 1)
    count = (num - 1) // (m + 1)
    ind = np.clip(count - 1, 0, n - 2)
    x_ind = (ind + 1) / (n + 1)
    xnew = (j + 1) / (m + 1)
    dx = 1.0 / (n + 1)
    eps = float(np.finfo(np.float32).eps)
    t = (xnew - x_ind) / (eps + dx)
    a = np.zeros((m, n), dtype=np.float64)
    np.add.at(a, (j, ind), 1.0 - t)
    np.add.at(a, (j, ind + 1), t)
    return a.astype(np.float32)


def _bit_reverse_perm(n: int) -> np.ndarray:
    bits = n.bit_length() - 1
    p = np.arange(n)
    r = np.zeros(n, dtype=np.int64)
    for b in range(bits):
        r |= ((p >> b) & 1) << (bits - 1 - b)
    return r


def _single_bit_reshape(x, jp):
    n, s = x.shape
    g = n // (2 * jp)
    xr = x.reshape(g, 2, jp, s)
    a = xr[:, 0]
    b = xr[:, 1]
    mn = jnp.minimum(a, b)[:, None]
    mx = jnp.maximum(a, b)[:, None]
    return jnp.concatenate([mn, mx], axis=1).reshape(n, s)


def _single_bit_roll(x, row, jp):
    low = (row & jp) == 0
    mn = jnp.minimum(x, jnp.roll(x, -jp, axis=0))
    mx = jnp.maximum(x, jnp.roll(x, jp, axis=0))
    return jnp.where(low, mn, mx)


def _bitonic_sort_bitrev(x):
    n = x.shape[0]
    bits = n.bit_length() - 1
    row = jax.lax.broadcasted_iota(jnp.int32, (n, 1), 0)

    def sgn_bit(a):
        return 1 << (bits - 1 - a)

    y = jnp.where((row & sgn_bit(1)) != 0, -x, x)
    for a in range(1, bits + 1):
        for b in range(a - 1, -1, -1):
            jp = 1 << (bits - 1 - b)
            if jp >= 8:
                y = _single_bit_reshape(y, jp)
            else:
                y = _single_bit_roll(y, row, jp)
        if a < bits:
            if a + 1 == bits:
                flip = (row & sgn_bit(a)) != 0
            else:
                flip = ((row & sgn_bit(a)) != 0) != ((row & sgn_bit(a + 1)) != 0)
            y = jnp.where(flip, -y, y)
    return y


def _body(x_ref, th_ref, ref_ref, w_ref, a_ref, out_ref):
    th = th_ref[...]
    inv_norm = jax.lax.rsqrt(jnp.sum(th * th, axis=1, keepdims=True))
    wn = th * inv_norm
    proj = jax.lax.dot_general(
        x_ref[0], wn, (((1,), (1,)), ((), ())),
        preferred_element_type=jnp.float32)
    srt = _bitonic_sort_bitrev(proj)
    w = w_ref[...]
    hi = jax.lax.Precision.HIGHEST
    wa = jax.lax.dot_general(
        w, a_ref[...], (((1,), (0,)), ((), ())),
        precision=hi, preferred_element_type=jnp.float32)
    red = jax.lax.dot_general(
        wa, srt, (((1,), (0,)), ((), ())),
        precision=hi, preferred_element_type=jnp.float32)
    cst = jax.lax.dot_general(
        w, ref_ref[...], (((1,), (0,)), ((), ())),
        precision=hi, preferred_element_type=jnp.float32)
    out_ref[...] = (cst - red)[None]


def kernel(X, theta_v, reference_pts, w):
    b, n, d = X.shape
    s = theta_v.shape[0]
    m = reference_pts.shape[0]
    a_np = _interp_matrix(n, m)
    rev = _bit_reverse_perm(n)
    a_mat = jnp.asarray(a_np[:, rev])

    s_blk = 256
    grid = (b, s // s_blk)

    out3 = pl.pallas_call(
        _body,
        grid=grid,
        in_specs=[
            pl.BlockSpec((1, n, d), lambda i, j: (i, 0, 0)),
            pl.BlockSpec((s_blk, d), lambda i, j: (j, 0)),
            pl.BlockSpec((m, s_blk), lambda i, j: (0, j)),
            pl.BlockSpec((1, m), lambda i, j: (0, 0)),
            pl.BlockSpec((m, n), lambda i, j: (0, 0)),
        ],
        out_specs=pl.BlockSpec((1, 1, s_blk), lambda i, j: (i, 0, j)),
        out_shape=jax.ShapeDtypeStruct((b, 1, s), jnp.float32),
        compiler_params=pltpu.CompilerParams(
            dimension_semantics=("parallel", "parallel"),
        ),
    )(X, theta_v, reference_pts, w, a_mat)
    return out3.reshape(b, s)
